# Initial kernel scaffold; baseline (speedup 1.0000x reference)
import jax
import jax.numpy as jnp
from jax.experimental import pallas as pl


def kernel(lengths, table):
    raise NotImplementedError("write your pallas kernel here")


# speedup vs baseline: 1.6272x; 1.6272x over previous
import functools

import jax
import jax.numpy as jnp
from jax import lax
from jax.experimental import pallas as pl
from jax.experimental.pallas import tpu as pltpu
from jax.experimental.pallas import tpu_sc as plsc

_B = 16384
_D = 20
_L = 16


def kernel(lengths, table):
    lengths = lengths.astype(jnp.int32)
    info = plsc.get_sparse_core_info()
    nw = info.num_cores * info.num_subcores
    b_per_w = _B // nw
    w_per_w = b_per_w * _D
    mesh = plsc.VectorSubcoreMesh(core_axis_name="c", subcore_axis_name="s")

    @functools.partial(
        pl.kernel,
        mesh=mesh,
        out_type=jax.ShapeDtypeStruct((_B * _D,), jnp.float32),
        scratch_types=[
            pltpu.VMEM((b_per_w,), jnp.int32),
            pltpu.VMEM((b_per_w,), jnp.int32),
            pltpu.VMEM((10, _D), jnp.float32),
            pltpu.VMEM((w_per_w,), jnp.float32),
        ],
        compiler_params=pltpu.CompilerParams(needs_layout_passes=False),
    )
    def sc_kernel(lengths_hbm, table_hbm, out_hbm, len_v, idx_v, table_v, out_v):
        wid = lax.axis_index("s") * info.num_cores + lax.axis_index("c")
        base = wid * b_per_w
        pltpu.sync_copy(lengths_hbm.at[pl.ds(base, b_per_w)], len_v)
        pltpu.sync_copy(table_hbm, table_v)

        def bin_body(c, carry):
            lv = len_v[pl.ds(c * _L, _L)]
            b = jnp.minimum(lv, 5)
            for t in (8, 16, 32, 64):
                b = b + jnp.where(lv >= t, 1, 0).astype(jnp.int32)
            idx_v[pl.ds(c * _L, _L)] = b
            return carry

        lax.fori_loop(0, b_per_w // _L, bin_body, 0, unroll=4)

        lane = lax.iota(jnp.int32, _L)
        rowp = []
        colp = []
        for k in range(5):
            p = lane + (_L * k)
            r = jnp.zeros((_L,), jnp.int32)
            for m in (1, 2, 3):
                r = r + jnp.where(p >= m * _D, 1, 0).astype(jnp.int32)
            rowp.append(r)
            colp.append(p - r * _D)

        def gather_body(g, carry):
            row0 = g * 4
            pos0 = g * 80
            for k in range(5):
                bins = plsc.load_gather(idx_v, [rowp[k] + row0])
                vals = plsc.load_gather(table_v, [bins, colp[k]])
                out_v[pl.ds(pos0 + _L * k, _L)] = vals
            return carry

        lax.fori_loop(0, b_per_w // 4, gather_body, 0, unroll=2)
        pltpu.sync_copy(out_v, out_hbm.at[pl.ds(base * _D, w_per_w)])

    return sc_kernel(lengths, table).reshape(_B, _D)


# speedup vs baseline: 1.8024x; 1.1077x over previous
import functools

import jax
import jax.numpy as jnp
from jax import lax
from jax.experimental import pallas as pl
from jax.experimental.pallas import tpu as pltpu
from jax.experimental.pallas import tpu_sc as plsc

_B = 16384
_D = 20
_L = 16


def kernel(lengths, table):
    lengths = lengths.astype(jnp.int32)
    info = plsc.get_sparse_core_info()
    nw = info.num_cores * info.num_subcores
    b_per_w = _B // nw
    mesh = plsc.VectorSubcoreMesh(core_axis_name="c", subcore_axis_name="s")

    @functools.partial(
        pl.kernel,
        mesh=mesh,
        out_type=jax.ShapeDtypeStruct((_B, _D), jnp.float32),
        scratch_types=[
            pltpu.VMEM((b_per_w,), jnp.int32),
            pltpu.VMEM((10, _D), jnp.float32),
            pltpu.VMEM((b_per_w, _D), jnp.float32),
        ],
        compiler_params=pltpu.CompilerParams(needs_layout_passes=False),
    )
    def sc_kernel(lengths_hbm, table_hbm, out_hbm, len_v, table_v, out_v):
        wid = lax.axis_index("s") * info.num_cores + lax.axis_index("c")
        base = wid * b_per_w
        pltpu.sync_copy(lengths_hbm.at[pl.ds(base, b_per_w)], len_v)
        pltpu.sync_copy(table_hbm, table_v)

        lane = lax.iota(jnp.int32, _L)
        zero = lane * 0
        cols = [zero + c for c in range(_D)]

        def body(c, carry):
            lv = len_v[pl.ds(c * _L, _L)]
            bv = jnp.minimum(lv, 5)
            for t in (8, 16, 32, 64):
                bv = bv + jnp.where(lv >= t, 1, 0).astype(jnp.int32)
            rows16 = lane + c * _L
            for col in range(_D):
                vals = plsc.load_gather(table_v, [bv, cols[col]])
                plsc.store_scatter(out_v, [rows16, cols[col]], vals)
            return carry

        lax.fori_loop(0, b_per_w // _L, body, 0, unroll=2)
        pltpu.sync_copy(out_v, out_hbm.at[pl.ds(base, b_per_w)])

    return sc_kernel(lengths, table)


# speedup vs baseline: 2.1429x; 1.1889x over previous
import functools

import jax
import jax.numpy as jnp
from jax import lax
from jax.experimental import pallas as pl
from jax.experimental.pallas import tpu as pltpu
from jax.experimental.pallas import tpu_sc as plsc

_B = 16384
_D = 20
_L = 16


def kernel(lengths, table):
    lengths = lengths.astype(jnp.int32)
    info = plsc.get_sparse_core_info()
    nw = info.num_cores * info.num_subcores
    b_per_w = _B // nw
    mesh = plsc.VectorSubcoreMesh(core_axis_name="c", subcore_axis_name="s")

    @functools.partial(
        pl.kernel,
        mesh=mesh,
        out_type=jax.ShapeDtypeStruct((_B, _D), jnp.float32),
        scratch_types=[
            pltpu.VMEM((b_per_w,), jnp.int32),
            pltpu.VMEM((10, _D), jnp.float32),
            pltpu.VMEM((b_per_w, _D), jnp.float32),
        ],
        compiler_params=pltpu.CompilerParams(needs_layout_passes=False),
    )
    def sc_kernel(lengths_hbm, table_hbm, out_hbm, len_v, table_v, out_v):
        wid = lax.axis_index("s") * info.num_cores + lax.axis_index("c")
        base = wid * b_per_w
        pltpu.sync_copy(lengths_hbm.at[pl.ds(base, b_per_w)], len_v)
        pltpu.sync_copy(table_hbm, table_v)

        lane = lax.iota(jnp.int32, _L)
        zero = lane * 0
        cols = [zero + c for c in range(_D)]

        @plsc.parallel_loop(0, b_per_w // _L, unroll=2)
        def body(c):
            lv = len_v[pl.ds(c * _L, _L)]
            bv = jnp.minimum(lv, 5)
            for t in (8, 16, 32, 64):
                bv = bv + jnp.where(lv >= t, 1, 0).astype(jnp.int32)
            rows16 = lane + c * _L
            for col in range(_D):
                vals = plsc.load_gather(table_v, [bv, cols[col]])
                plsc.store_scatter(out_v, [rows16, cols[col]], vals)

        pltpu.sync_copy(out_v, out_hbm.at[pl.ds(base, b_per_w)])

    return sc_kernel(lengths, table)
